# Initial kernel scaffold; baseline (speedup 1.0000x reference)
import jax
import jax.numpy as jnp
from jax.experimental import pallas as pl


def kernel(x_drugs, x_prots, dp_edge_index, pp_edge_index, dd_pair_index, prot_emb, W1_d_self, W1_p2d, W1_d2p, W1_p_self, W1_p2p, b1_d, b1_p, W_res, b_res, Wp1, bp1, Wp2, bp2, Wp3, bp3):
    raise NotImplementedError("write your pallas kernel here")


# speedup vs baseline: 2.5819x; 2.5819x over previous
import functools

import jax
import jax.numpy as jnp
from jax import lax
from jax.experimental import pallas as pl
from jax.experimental.pallas import tpu as pltpu
from jax.experimental.pallas import tpu_sc as plsc

N = 10000
H = 256
HH = 128
CHUNK = 128
NCH = 79
EPW = NCH * CHUNK
EPAD = 16 * EPW
NACC = 10112
ZROWS = NACC // 16
OROWS = N // 16
B = 4096
ROWB = 1000

_SC_MESH = plsc.VectorSubcoreMesh(core_axis_name="c", subcore_axis_name="s")


def _msg_mm_body(h_ref, w_ref, o_ref):
    res = jnp.dot(h_ref[...], w_ref[...], preferred_element_type=jnp.float32)
    o_ref[0] = res[:, :HH]
    o_ref[1] = res[:, HH:]


def _msg_mm(h, w):
    return pl.pallas_call(
        _msg_mm_body,
        grid=(N // ROWB,),
        in_specs=[
            pl.BlockSpec((ROWB, H), lambda i: (i, 0)),
            pl.BlockSpec((H, H), lambda i: (0, 0)),
        ],
        out_specs=pl.BlockSpec((2, ROWB, HH), lambda i: (0, i, 0)),
        out_shape=jax.ShapeDtypeStruct((2, N, HH), jnp.float32),
    )(h, w)


def _combine(h, w, b, msgs, prev=None):
    nm = len(msgs)
    has_prev = prev is not None

    def body(*refs):
        h_ref, w_ref, b_ref = refs[0], refs[1], refs[2]
        msg_refs = refs[3:3 + nm]
        o_ref = refs[-1]
        acc = jnp.dot(h_ref[...], w_ref[...], preferred_element_type=jnp.float32)
        acc = acc + b_ref[...]
        for m in msg_refs:
            acc = acc + jnp.concatenate([m[0], m[1]], axis=1)
        acc = jnp.maximum(acc, 0.0)
        if has_prev:
            acc = refs[3 + nm][...] + acc
        o_ref[...] = acc

    in_specs = [
        pl.BlockSpec((ROWB, H), lambda i: (i, 0)),
        pl.BlockSpec((H, H), lambda i: (0, 0)),
        pl.BlockSpec((1, H), lambda i: (0, 0)),
    ]
    args = [h, w, b.reshape(1, H)]
    for m in msgs:
        in_specs.append(pl.BlockSpec((2, ROWB, HH), lambda i: (0, i, 0)))
        args.append(m)
    if has_prev:
        in_specs.append(pl.BlockSpec((ROWB, H), lambda i: (i, 0)))
        args.append(prev)
    return pl.pallas_call(
        body,
        grid=(N // ROWB,),
        in_specs=in_specs,
        out_specs=pl.BlockSpec((ROWB, H), lambda i: (i, 0)),
        out_shape=jax.ShapeDtypeStruct((N, H), jnp.float32),
    )(*args)


def _mlp_body(g_ref, w1a_ref, w1b_ref, b1_ref, w2_ref, b2_ref, w3_ref, b3_ref,
              o_ref):
    h = jnp.dot(g_ref[0], w1a_ref[...], preferred_element_type=jnp.float32)
    h = h + jnp.dot(g_ref[1], w1b_ref[...], preferred_element_type=jnp.float32)
    h = jnp.maximum(h + b1_ref[...], 0.0)
    h = jnp.dot(h, w2_ref[...], preferred_element_type=jnp.float32)
    h = jnp.maximum(h + b2_ref[...], 0.0)
    o_ref[...] = (jnp.dot(h, w3_ref[...], preferred_element_type=jnp.float32)
                  + b3_ref[...])


def _mlp(g, w1a, w1b, b1, w2, b2, w3p, b3p):
    blk = 1024
    full = lambda r, c: pl.BlockSpec((r, c), lambda i: (0, 0))
    return pl.pallas_call(
        _mlp_body,
        grid=(B // blk,),
        in_specs=[
            pl.BlockSpec((2, blk, H), lambda i: (0, i, 0)),
            full(H, 128), full(H, 128), full(1, 128),
            full(128, 64), full(1, 64),
            full(64, 128), full(1, 128),
        ],
        out_specs=pl.BlockSpec((blk, 128), lambda i: (i, 0)),
        out_shape=jax.ShapeDtypeStruct((B, 128), jnp.float32),
    )(g, w1a, w1b, b1, w2, b2, w3p, b3p)


def _seg_body(M_hbm, src_hbm, dst_hbm, z_hbm, out_hbm,
              src_v, dst_v, rows_v, acc, sem):
    c = lax.axis_index("c")
    s = lax.axis_index("s")
    pltpu.sync_copy(src_hbm.at[s], src_v)
    pltpu.sync_copy(dst_hbm.at[s], dst_v)
    pltpu.sync_copy(z_hbm, acc.at[pl.ds(s * ZROWS, ZROWS)])
    plsc.subcore_barrier()
    Mh = M_hbm.at[c]

    def body(j, carry):
        pltpu.async_copy(Mh.at[src_v.at[j]], rows_v, sem).wait()
        pltpu.sync_copy(rows_v, acc.at[dst_v.at[j]], add=True)
        return carry

    lax.fori_loop(0, NCH, body, 0)
    plsc.subcore_barrier()
    pltpu.sync_copy(acc.at[pl.ds(s * ZROWS, ZROWS)],
                    out_hbm.at[c, pl.ds(s * ZROWS, ZROWS)])


_segsum = pl.kernel(
    _seg_body,
    out_type=jax.ShapeDtypeStruct((2, NACC, HH), jnp.float32),
    mesh=_SC_MESH,
    scratch_types=[
        pltpu.VMEM((NCH, CHUNK), jnp.int32),
        pltpu.VMEM((NCH, CHUNK), jnp.int32),
        pltpu.VMEM((CHUNK, HH), jnp.float32),
        pltpu.VMEM_SHARED((NACC, HH), jnp.float32),
        pltpu.SemaphoreType.DMA,
    ],
)


def _pairgather_body(hd_hbm, i0_hbm, i1_hbm, out_hbm, idx_v, rows_v, sem):
    c = lax.axis_index("c")
    s = lax.axis_index("s")
    w = s * 2 + c
    pltpu.sync_copy(i0_hbm.at[w], idx_v)
    pltpu.async_copy(hd_hbm.at[idx_v], rows_v, sem).wait()
    pltpu.sync_copy(rows_v, out_hbm.at[0, pl.ds(w * 128, 128)])
    pltpu.sync_copy(i1_hbm.at[w], idx_v)
    pltpu.async_copy(hd_hbm.at[idx_v], rows_v, sem).wait()
    pltpu.sync_copy(rows_v, out_hbm.at[1, pl.ds(w * 128, 128)])


_pairgather = pl.kernel(
    _pairgather_body,
    out_type=jax.ShapeDtypeStruct((2, B, H), jnp.float32),
    mesh=_SC_MESH,
    scratch_types=[
        pltpu.VMEM((128,), jnp.int32),
        pltpu.VMEM((128, H), jnp.float32),
        pltpu.SemaphoreType.DMA,
    ],
)


def _prep_edges(src, dst):
    npad = EPAD - src.shape[0]
    src = jnp.concatenate([src, jnp.zeros((npad,), jnp.int32)])
    dump = N + (jnp.arange(npad, dtype=jnp.int32) % (NACC - N))
    dst = jnp.concatenate([dst, dump])
    return src.reshape(16, NCH, CHUNK), dst.reshape(16, NCH, CHUNK)


def kernel(x_drugs, x_prots, dp_edge_index, pp_edge_index, dd_pair_index,
           prot_emb, W1_d_self, W1_p2d, W1_d2p, W1_p_self, W1_p2p, b1_d, b1_p,
           W_res, b_res, Wp1, bp1, Wp2, bp2, Wp3, bp3):
    dp = dp_edge_index.astype(jnp.int32)
    pp = pp_edge_index.astype(jnp.int32)
    p2d = _prep_edges(dp[1], dp[0])
    d2p = _prep_edges(dp[0], dp[1])
    p2p = _prep_edges(pp[0], pp[1])
    z = jnp.zeros((ZROWS, HH), jnp.float32)

    h_p = jnp.concatenate([prot_emb, x_prots], axis=1)
    h_d = x_drugs

    layers = [(W1_d_self, W1_p2d, W1_d2p, W1_p_self, W1_p2p, b1_d, b1_p, False)]
    for i in range(2):
        layers.append((W_res[i, 0], W_res[i, 1], W_res[i, 2], W_res[i, 3],
                       W_res[i, 4], b_res[i, 0], b_res[i, 1], True))

    for (W_ds, W_p2d, W_d2p, W_ps, W_p2p, b_d, b_p, resid) in layers:
        Mp2d = _msg_mm(h_p, W_p2d)
        Md2p = _msg_mm(h_d, W_d2p)
        Mp2p = _msg_mm(h_p, W_p2p)
        msg_p2d = _segsum(Mp2d, p2d[0], p2d[1], z)
        msg_d2p = _segsum(Md2p, d2p[0], d2p[1], z)
        msg_p2p = _segsum(Mp2p, p2p[0], p2p[1], z)
        h_d_new = _combine(h_d, W_ds, b_d, [msg_p2d],
                           prev=h_d if resid else None)
        h_p_new = _combine(h_p, W_ps, b_p, [msg_d2p, msg_p2p],
                           prev=h_p if resid else None)
        h_d, h_p = h_d_new, h_p_new

    i0 = dd_pair_index[0].astype(jnp.int32).reshape(32, 128)
    i1 = dd_pair_index[1].astype(jnp.int32).reshape(32, 128)
    g = _pairgather(h_d, i0, i1)

    w3p = jnp.pad(Wp3, ((0, 0), (0, 127)))
    b3p = jnp.pad(bp3.reshape(1, 1), ((0, 0), (0, 127)))
    out = _mlp(g, Wp1[:H], Wp1[H:], bp1.reshape(1, 128), Wp2,
               bp2.reshape(1, 64), w3p, b3p)
    return out[:, :1, None]
